# baseline (speedup 1.0000x reference)
import functools

import jax
import jax.numpy as jnp
from jax.experimental import pallas as pl
from jax.experimental.pallas import tpu as pltpu

_CORES = 2
_TILES = 4


def _mm_kernel(x_hbm, w_ref, b_ref, o_hbm, xbuf, obuf, in_sems, out_sems, *, tile):
    core = pl.program_id(0)
    base = core * (_TILES * tile)

    wb = w_ref[...].astype(jnp.bfloat16)
    bias_row = b_ref[...]

    def in_copy(t, slot):
        return pltpu.make_async_copy(
            x_hbm.at[pl.ds(base + t * tile, tile), :],
            xbuf.at[slot],
            in_sems.at[slot],
        )

    def out_copy(t, slot):
        return pltpu.make_async_copy(
            obuf.at[slot],
            o_hbm.at[pl.ds(base + t * tile, tile), :],
            out_sems.at[slot],
        )

    in_copy(0, 0).start()
    in_copy(1, 1).start()
    for t in range(_TILES):
        slot = t % 2
        in_copy(t, slot).wait()
        xb = xbuf[slot].astype(jnp.bfloat16)
        y = jax.lax.dot_general(
            xb,
            wb,
            dimension_numbers=(((1,), (1,)), ((), ())),
            preferred_element_type=jnp.float32,
        )
        if t >= 2:
            out_copy(t - 2, slot).wait()
        obuf[slot] = y + bias_row
        out_copy(t, slot).start()
        if t + 2 < _TILES:
            in_copy(t + 2, slot).start()
    out_copy(_TILES - 2, _TILES % 2).wait()
    out_copy(_TILES - 1, (_TILES - 1) % 2).wait()


def kernel(x, weight, bias):
    B, D_in = x.shape
    D_out, D_in_w = weight.shape
    assert D_in == D_in_w and bias.shape == (D_out,)
    assert B % (_CORES * _TILES) == 0
    tile = B // (_CORES * _TILES)

    b2 = bias.reshape(1, D_out)
    return pl.pallas_call(
        functools.partial(_mm_kernel, tile=tile),
        grid=(_CORES,),
        in_specs=[
            pl.BlockSpec(memory_space=pltpu.MemorySpace.HBM),
            pl.BlockSpec((D_out, D_in), lambda i: (0, 0)),
            pl.BlockSpec((1, D_out), lambda i: (0, 0)),
        ],
        out_specs=pl.BlockSpec(memory_space=pltpu.MemorySpace.HBM),
        out_shape=jax.ShapeDtypeStruct((B, D_out), x.dtype),
        scratch_shapes=[
            pltpu.VMEM((2, B // (_CORES * _TILES), D_in), jnp.float32),
            pltpu.VMEM((2, B // (_CORES * _TILES), D_out), jnp.float32),
            pltpu.SemaphoreType.DMA((2,)),
            pltpu.SemaphoreType.DMA((2,)),
        ],
        compiler_params=pltpu.CompilerParams(
            dimension_semantics=("parallel",),
            vmem_limit_bytes=64 * 1024 * 1024,
        ),
    )(x, weight, b2)


# speedup vs baseline: 1.1924x; 1.1924x over previous
import jax
import jax.numpy as jnp
from jax.experimental import pallas as pl
from jax.experimental.pallas import tpu as pltpu

_TM = 1024


def _linear_kernel(x_ref, w_ref, b_ref, o_ref):
    xb = x_ref[...].astype(jnp.bfloat16)
    wb = w_ref[...].astype(jnp.bfloat16)
    y = jax.lax.dot_general(
        xb,
        wb,
        dimension_numbers=(((1,), (1,)), ((), ())),
        preferred_element_type=jnp.float32,
    )
    o_ref[...] = y + b_ref[...]


def kernel(x, weight, bias):
    B, D_in = x.shape
    D_out, D_in_w = weight.shape
    assert D_in == D_in_w and bias.shape == (D_out,)
    assert B % _TM == 0

    b2 = bias.reshape(1, D_out)
    return pl.pallas_call(
        _linear_kernel,
        grid=(B // _TM,),
        in_specs=[
            pl.BlockSpec((_TM, D_in), lambda i: (i, 0)),
            pl.BlockSpec((D_out, D_in), lambda i: (0, 0),
                         pipeline_mode=pl.Buffered(buffer_count=1)),
            pl.BlockSpec((1, D_out), lambda i: (0, 0),
                         pipeline_mode=pl.Buffered(buffer_count=1)),
        ],
        out_specs=pl.BlockSpec((_TM, D_out), lambda i: (i, 0)),
        out_shape=jax.ShapeDtypeStruct((B, D_out), x.dtype),
        compiler_params=pltpu.CompilerParams(
            dimension_semantics=("parallel",),
            vmem_limit_bytes=64 * 1024 * 1024,
        ),
    )(x, weight, b2)


# speedup vs baseline: 1.1958x; 1.0028x over previous
import jax
import jax.numpy as jnp
from jax.experimental import pallas as pl
from jax.experimental.pallas import tpu as pltpu

_TM = 1024


def _linear_kernel(x_ref, w_ref, b_ref, o_ref):
    xb = x_ref[...].astype(jnp.bfloat16)
    wb = w_ref[...].astype(jnp.bfloat16)
    y = jax.lax.dot_general(
        xb,
        wb,
        dimension_numbers=(((1,), (1,)), ((), ())),
        preferred_element_type=jnp.float32,
    )
    o_ref[...] = y + b_ref[...]


def kernel(x, weight, bias):
    B, D_in = x.shape
    D_out, D_in_w = weight.shape
    assert D_in == D_in_w and bias.shape == (D_out,)
    assert B % _TM == 0

    b2 = bias.reshape(1, D_out)
    return pl.pallas_call(
        _linear_kernel,
        grid=(B // _TM,),
        in_specs=[
            pl.BlockSpec((_TM, D_in), lambda i: (i, 0)),
            pl.BlockSpec((D_out, D_in), lambda i: (0, 0)),
            pl.BlockSpec((1, D_out), lambda i: (0, 0)),
        ],
        out_specs=pl.BlockSpec((_TM, D_out), lambda i: (i, 0)),
        out_shape=jax.ShapeDtypeStruct((B, D_out), x.dtype),
        compiler_params=pltpu.CompilerParams(
            dimension_semantics=("parallel",),
            vmem_limit_bytes=64 * 1024 * 1024,
        ),
    )(x, weight, b2)
